# baseline (speedup 1.0000x reference)
import functools

import jax
import jax.numpy as jnp
import numpy as np
from jax import lax
from jax.experimental import pallas as pl
from jax.experimental.pallas import tpu as pltpu
from jax.experimental.pallas import tpu_sc as plsc

_B = 16384
_F = 26
_D = 16
_S = 32
_NW = 32
_NHALF = 2
_BH = _B // _NHALF
_TOT = _BH * _S
_SUB = 128
_NROWS = _TOT // _SUB
_ROWS_PER_W = _NROWS // _NW
_K = 16
_CH = 32


def _sc_gather_body(dtab, wtab, idx_hbm, eout, wout, idx_v, rows_v, sem):
    c = lax.axis_index("c")
    s = lax.axis_index("s")
    wid = s * 2 + c
    row0 = wid * _ROWS_PER_W
    for tab, out in ((dtab, eout), (wtab, wout)):
        for half in range(_ROWS_PER_W // _CH):
            base = row0 + half * _CH
            pltpu.sync_copy(idx_hbm.at[pl.ds(base * _SUB, _CH * _SUB)], idx_v)
            for g in range(_CH // _K):
                copies = []
                for j in range(_K):
                    r = g * _K + j
                    copies.append(
                        pltpu.async_copy(
                            tab.at[idx_v.at[pl.ds(r * _SUB, _SUB)]],
                            rows_v.at[r], sem)
                    )
                for cp in copies:
                    cp.wait()
            pltpu.sync_copy(rows_v, out.at[pl.ds(base, _CH)])


@functools.partial(
    pl.kernel,
    mesh=plsc.VectorSubcoreMesh(core_axis_name="c", subcore_axis_name="s"),
    compiler_params=pltpu.CompilerParams(use_tc_tiling_on_sc=False),
    out_type=(jax.ShapeDtypeStruct((_NROWS, _SUB, _D), jnp.float32),
              jax.ShapeDtypeStruct((_NROWS, _SUB, _D), jnp.float32)),
    scratch_types=[
        pltpu.VMEM((_CH * _SUB,), jnp.int32),
        pltpu.VMEM((_CH, _SUB, _D), jnp.float32),
        pltpu.SemaphoreType.DMA,
    ],
)
def _sc_gather(dtab, wtab, idx_hbm, eout, wout, idx_v, rows_v, sem):
    _sc_gather_body(dtab, wtab, idx_hbm, eout, wout, idx_v, rows_v, sem)


def _dense_body(e_ref, we_ref, w1a_ref, w1g_ref, w2_ref, w3_ref, w4_ref,
                lrw_ref, b1_ref, b2_ref, b3_ref, bo_ref, out_ref):
    bb = e_ref.shape[0] * 128 // (_S * _D)
    v4 = e_ref[...].reshape(bb, 4, 128)
    xp = jnp.concatenate([v4[:, q, :] for q in range(4)], axis=1)
    e3 = xp.reshape(bb, _S, _D)[:, :_F, :]
    gram = lax.dot_general(
        e3, e3, (((2,), (2,)), ((0,), (0,))),
        preferred_element_type=jnp.float32)
    gflat = gram.reshape(bb, _F * _F)
    h = xp @ w1a_ref[...] + gflat @ w1g_ref[...] + b1_ref[...]
    h = jnp.maximum(h, 0.0)
    h = jnp.maximum(h @ w2_ref[...] + b2_ref[...], 0.0)
    h = jnp.maximum(h @ w3_ref[...] + b3_ref[...], 0.0)
    we4 = we_ref[...].reshape(bb, 4, 128)
    wep = jnp.concatenate([we4[:, q, :] for q in range(4)], axis=1)
    logit = h @ w4_ref[...] + wep @ lrw_ref[...] + bo_ref[...]
    out_ref[...] = jax.nn.sigmoid(logit)


def _dense_call(e, we, w1ap, w1g, w2, w3, w4, lrwp, b1, b2, b3, bo, bB=512):
    grid = _BH // bB
    sd = _S * _D
    rpb = bB * sd // 128
    return pl.pallas_call(
        _dense_body,
        grid=(grid,),
        in_specs=[
            pl.BlockSpec((rpb, 128), lambda i: (i, 0)),
            pl.BlockSpec((rpb, 128), lambda i: (i, 0)),
            pl.BlockSpec((sd, 512), lambda i: (0, 0)),
            pl.BlockSpec((_F * _F, 512), lambda i: (0, 0)),
            pl.BlockSpec((512, 512), lambda i: (0, 0)),
            pl.BlockSpec((512, 512), lambda i: (0, 0)),
            pl.BlockSpec((512, 1), lambda i: (0, 0)),
            pl.BlockSpec((sd, 1), lambda i: (0, 0)),
            pl.BlockSpec((1, 512), lambda i: (0, 0)),
            pl.BlockSpec((1, 512), lambda i: (0, 0)),
            pl.BlockSpec((1, 512), lambda i: (0, 0)),
            pl.BlockSpec((1, 1), lambda i: (0, 0)),
        ],
        out_specs=pl.BlockSpec((bB, 1), lambda i: (i, 0)),
        out_shape=jax.ShapeDtypeStruct((_BH, 1), jnp.float32),
    )(e, we, w1ap, w1g, w2, w3, w4, lrwp, b1, b2, b3, bo)


def kernel(inputs, deep_table, wide_table, W1, b1, W2, b2, W3, b3, W4, b4, lr_W, lr_b):
    idx = inputs.astype(jnp.int32)
    idxp = jnp.concatenate(
        [idx, jnp.zeros((_B, _S - _F), jnp.int32)], axis=1)
    idxp = idxp.reshape(_NHALF, _TOT)

    iu0, iu1 = np.triu_indices(_F, k=1)
    npad = _S * _D - _F * _D
    w1ap = jnp.concatenate(
        [W1[: _F * _D], jnp.zeros((npad, 512), jnp.float32)], axis=0)
    lrwp = jnp.concatenate(
        [lr_W, jnp.zeros((npad, 1), jnp.float32)], axis=0)
    w1g = jnp.zeros((_F * _F, 512), jnp.float32).at[iu0 * _F + iu1].set(W1[_F * _D :])
    bo = (b4 + lr_b).reshape(1, 1)
    b1r, b2r, b3r = b1.reshape(1, 512), b2.reshape(1, 512), b3.reshape(1, 512)

    nlr = _TOT * _D // 128
    ews = [_sc_gather(deep_table, wide_table, idxp[h]) for h in range(_NHALF)]
    outs = [
        _dense_call(e.reshape(nlr, 128), we.reshape(nlr, 128),
                    w1ap, w1g, W2, W3, W4, lrwp, b1r, b2r, b3r, bo)
        for (e, we) in ews
    ]
    return jnp.concatenate(outs, axis=0)


# speedup vs baseline: 1.8229x; 1.8229x over previous
import functools

import jax
import jax.numpy as jnp
import numpy as np
from jax import lax
from jax.experimental import pallas as pl
from jax.experimental.pallas import tpu as pltpu
from jax.experimental.pallas import tpu_sc as plsc

_B = 16384
_F = 26
_D = 16
_S = 32
_NW = 32
_NHALF = 2
_BH = _B // _NHALF
_TOT = _BH * _S
_SUB = 128
_NROWS = _TOT // _SUB
_ROWS_PER_W = _NROWS // _NW
_K = 16
_CH = 32


def _sc_gather_body(dtab, wtab, idx_hbm, eout, wout, idx_v, rows_v, sem):
    c = lax.axis_index("c")
    s = lax.axis_index("s")
    wid = s * 2 + c
    row0 = wid * _ROWS_PER_W
    for tab, out in ((dtab, eout), (wtab, wout)):
        for half in range(_ROWS_PER_W // _CH):
            base = row0 + half * _CH
            pltpu.sync_copy(idx_hbm.at[pl.ds(base * _SUB, _CH * _SUB)], idx_v)
            for g in range(_CH // _K):
                copies = []
                for j in range(_K):
                    r = g * _K + j
                    copies.append(
                        pltpu.async_copy(
                            tab.at[idx_v.at[pl.ds(r * _SUB, _SUB)]],
                            rows_v.at[r], sem)
                    )
                for cp in copies:
                    cp.wait()
            pltpu.sync_copy(rows_v, out.at[pl.ds(base, _CH)])


@functools.partial(
    pl.kernel,
    mesh=plsc.VectorSubcoreMesh(core_axis_name="c", subcore_axis_name="s"),
    compiler_params=pltpu.CompilerParams(use_tc_tiling_on_sc=False),
    out_type=(jax.ShapeDtypeStruct((_NROWS, _SUB, _D), jnp.float32),
              jax.ShapeDtypeStruct((_NROWS, _SUB, _D), jnp.float32)),
    scratch_types=[
        pltpu.VMEM((_CH * _SUB,), jnp.int32),
        pltpu.VMEM((_CH, _SUB, _D), jnp.float32),
        pltpu.SemaphoreType.DMA,
    ],
)
def _sc_gather(dtab, wtab, idx_hbm, eout, wout, idx_v, rows_v, sem):
    _sc_gather_body(dtab, wtab, idx_hbm, eout, wout, idx_v, rows_v, sem)


def _dense_body(e_ref, we_ref, w1a_ref, w1g_ref, w2_ref, w3_ref, w4_ref,
                lrw_ref, b1_ref, b2_ref, b3_ref, bo_ref, out_ref):
    bb = e_ref.shape[0] * 128 // (_S * _D)
    v4 = e_ref[...].reshape(bb, 4, 128)
    xp = jnp.concatenate([v4[:, q, :] for q in range(4)], axis=1)
    e3 = xp.reshape(bb, _S, _D)[:, :_F, :]
    gram = lax.dot_general(
        e3, e3, (((2,), (2,)), ((0,), (0,))),
        preferred_element_type=jnp.float32)
    gflat = gram.reshape(bb, _F * _F)
    h = xp @ w1a_ref[...] + gflat @ w1g_ref[...] + b1_ref[...]
    h = jnp.maximum(h, 0.0)
    h = jnp.maximum(h @ w2_ref[...] + b2_ref[...], 0.0)
    h = jnp.maximum(h @ w3_ref[...] + b3_ref[...], 0.0)
    we4 = we_ref[...].reshape(bb, 4, 128)
    wep = jnp.concatenate([we4[:, q, :] for q in range(4)], axis=1)
    logit = h @ w4_ref[...] + wep @ lrw_ref[...] + bo_ref[...]
    out_ref[...] = jax.nn.sigmoid(logit)


def _dense_call(e, we, w1ap, w1g, w2, w3, w4, lrwp, b1, b2, b3, bo, bB=512):
    grid = _BH // bB
    sd = _S * _D
    rpb = bB * sd // 128
    return pl.pallas_call(
        _dense_body,
        grid=(grid,),
        in_specs=[
            pl.BlockSpec((rpb, 128), lambda i: (i, 0)),
            pl.BlockSpec((rpb, 128), lambda i: (i, 0)),
            pl.BlockSpec((sd, 512), lambda i: (0, 0)),
            pl.BlockSpec((_F * _F, 512), lambda i: (0, 0)),
            pl.BlockSpec((512, 512), lambda i: (0, 0)),
            pl.BlockSpec((512, 512), lambda i: (0, 0)),
            pl.BlockSpec((512, 1), lambda i: (0, 0)),
            pl.BlockSpec((sd, 1), lambda i: (0, 0)),
            pl.BlockSpec((1, 512), lambda i: (0, 0)),
            pl.BlockSpec((1, 512), lambda i: (0, 0)),
            pl.BlockSpec((1, 512), lambda i: (0, 0)),
            pl.BlockSpec((1, 1), lambda i: (0, 0)),
        ],
        out_specs=pl.BlockSpec((bB, 1), lambda i: (i, 0)),
        out_shape=jax.ShapeDtypeStruct((_BH, 1), jnp.float32),
    )(e, we, w1ap, w1g, w2, w3, w4, lrwp, b1, b2, b3, bo)


def kernel(inputs, deep_table, wide_table, W1, b1, W2, b2, W3, b3, W4, b4, lr_W, lr_b):
    idx = inputs.astype(jnp.int32)
    nv = deep_table.shape[0]
    pad = (jnp.arange(_B * (_S - _F), dtype=jnp.int32) % nv).reshape(_B, _S - _F)
    idxp = jnp.concatenate([idx, pad], axis=1).reshape(_NHALF, _TOT)

    iu0, iu1 = np.triu_indices(_F, k=1)
    npad = _S * _D - _F * _D
    w1ap = jnp.concatenate(
        [W1[: _F * _D], jnp.zeros((npad, 512), jnp.float32)], axis=0)
    lrwp = jnp.concatenate(
        [lr_W, jnp.zeros((npad, 1), jnp.float32)], axis=0)
    w1g = jnp.zeros((_F * _F, 512), jnp.float32).at[iu0 * _F + iu1].set(W1[_F * _D :])
    bo = (b4 + lr_b).reshape(1, 1)
    b1r, b2r, b3r = b1.reshape(1, 512), b2.reshape(1, 512), b3.reshape(1, 512)

    nl = _NROWS * _SUB * _D // 128
    ews = [_sc_gather(deep_table, wide_table, idxp[h]) for h in range(_NHALF)]
    outs = [
        _dense_call(e.reshape(nl, 128), we.reshape(nl, 128),
                    w1ap, w1g, W2, W3, W4, lrwp, b1r, b2r, b3r, bo)
        for (e, we) in ews
    ]
    return jnp.concatenate(outs, axis=0)
